# Initial kernel scaffold; baseline (speedup 1.0000x reference)
import jax
import jax.numpy as jnp
from jax.experimental import pallas as pl


def kernel(x, edge_index, edge_attr, u, batch, W1, b1, W2, b2):
    raise NotImplementedError("write your pallas kernel here")


# speedup vs baseline: 29.8432x; 29.8432x over previous
import functools

import jax
import jax.numpy as jnp
from jax import lax
from jax.experimental import pallas as pl
from jax.experimental.pallas import tpu as pltpu
from jax.experimental.pallas import tpu_sc as plsc

N_NODES = 100000
N_EDGES = 3200000
N_EDGE_F = 16
GLOBAL_F = 64
NUM_GRAPHS = 512
HIDDEN = 8

NC = 2
NS = 16
NW = NC * NS
BLK = 128
KB = 16
NB = N_EDGES // BLK


def _sc_body(src_hbm, attr_hbm, batch_hbm, sums_out, cnt_out,
             idx_v, seg_v, attr_v, cnt16, z2, acc_sh, gsem, ssem):
    cid = lax.axis_index("c")
    sid = lax.axis_index("s")
    wid = sid * NC + cid

    iota = lax.iota(jnp.int32, 16)
    ones = jnp.ones((16,), jnp.float32)
    zeros = jnp.zeros((16,), jnp.float32)

    def _zero(r, _):
        cnt16[pl.ds(r * 16, 16)] = zeros
        z2[r, :] = zeros
        return 0
    lax.fori_loop(0, NUM_GRAPHS, _zero, 0)

    @pl.when(sid == 0)
    def _():
        pltpu.sync_copy(z2, acc_sh)

    plsc.subcore_barrier()

    nsb = NB // 8
    s0 = (nsb * wid) // NW
    s1 = (nsb * (wid + 1)) // NW
    b0 = s0 * 8
    b1 = s1 * 8

    def process(blk0, kb):
        pltpu.sync_copy(src_hbm.at[pl.ds(blk0, kb)], idx_v.at[pl.ds(0, kb)])
        gd = [pltpu.async_copy(batch_hbm.at[idx_v.at[j]], seg_v.at[j], gsem)
              for j in range(kb)]
        pltpu.sync_copy(attr_hbm.at[pl.ds(blk0, kb)], attr_v.at[pl.ds(0, kb)])
        for d in gd:
            d.wait()
        sd = [pltpu.async_copy(attr_v.at[j], acc_sh.at[seg_v.at[j]], ssem,
                               add=True)
              for j in range(kb)]
        for j in range(kb):
            for t in range(BLK // 16):
                s = seg_v[j, pl.ds(t * 16, 16)]
                plsc.addupdate_scatter(cnt16, [s * 16 + iota], ones)
        for d in sd:
            d.wait()

    nf = (b1 - b0) // KB

    def chunk_body(i, _):
        process(b0 + i * KB, KB)
        return 0
    lax.fori_loop(0, nf, chunk_body, 0)

    def tail_body(b, _):
        process(b, 8)
        return 0
    lax.fori_loop(0, (b1 - b0 - nf * KB) // 8,
                  lambda i, _: tail_body(b0 + nf * KB + i * 8, 0), 0)

    plsc.subcore_barrier()

    pltpu.sync_copy(cnt16, cnt_out.at[wid])

    @pl.when(sid == 0)
    def _():
        pltpu.sync_copy(acc_sh, sums_out.at[cid])


_sc_seg = functools.partial(
    pl.kernel,
    out_type=[
        jax.ShapeDtypeStruct((NC, NUM_GRAPHS, N_EDGE_F), jnp.float32),
        jax.ShapeDtypeStruct((NW, NUM_GRAPHS * N_EDGE_F), jnp.float32),
    ],
    mesh=plsc.VectorSubcoreMesh(core_axis_name="c", subcore_axis_name="s"),
    scratch_types=[
        pltpu.VMEM((KB, BLK), jnp.int32),
        pltpu.VMEM((KB, BLK), jnp.int32),
        pltpu.VMEM((KB, BLK, N_EDGE_F), jnp.float32),
        pltpu.VMEM((NUM_GRAPHS * N_EDGE_F,), jnp.float32),
        pltpu.VMEM((NUM_GRAPHS, N_EDGE_F), jnp.float32),
        pltpu.VMEM_SHARED((NUM_GRAPHS, N_EDGE_F), jnp.float32),
        pltpu.SemaphoreType.DMA,
        pltpu.SemaphoreType.DMA,
    ],
    compiler_params=pltpu.CompilerParams(needs_layout_passes=False,
                                         use_tc_tiling_on_sc=False),
)(_sc_body)


def _mlp_body(sums_ref, cnt_ref, u_ref, w1u_ref, w1m_ref, b1_ref, w2_ref,
              b2_ref, o_ref):
    sums = sums_ref[0] + sums_ref[1]
    counts = jnp.sum(cnt_ref[...].reshape(NW, NUM_GRAPHS, N_EDGE_F),
                     axis=(0, 2))
    mean = sums / jnp.maximum(counts, 1.0)[:, None]
    h = jnp.dot(u_ref[...], w1u_ref[...], preferred_element_type=jnp.float32)
    h = h + jnp.dot(mean, w1m_ref[...], preferred_element_type=jnp.float32)
    h = jnp.maximum(h + b1_ref[...], 0.0)
    o = jnp.dot(h, w2_ref[...], preferred_element_type=jnp.float32)
    o_ref[...] = o + b2_ref[...]


def _mlp(sums_p, cnt_p, u, w1u_t, w1m_t, b1, w2_t, b2):
    return pl.pallas_call(
        _mlp_body,
        out_shape=jax.ShapeDtypeStruct((NUM_GRAPHS, GLOBAL_F), jnp.float32),
    )(sums_p, cnt_p, u, w1u_t, w1m_t, b1, w2_t, b2)


def kernel(x, edge_index, edge_attr, u, batch, W1, b1, W2, b2):
    src = edge_index[0].astype(jnp.int32).reshape(NB, BLK)
    attr = edge_attr.reshape(NB, BLK, N_EDGE_F)
    batch32 = batch.astype(jnp.int32)
    sums_p, cnt_p = _sc_seg(src, attr, batch32)
    w1u_t = W1[:, :GLOBAL_F].T
    w1m_t = W1[:, GLOBAL_F:].T
    w2_t = W2.T
    return _mlp(sums_p, cnt_p, u, w1u_t, w1m_t,
                b1.reshape(1, HIDDEN), w2_t, b2.reshape(1, GLOBAL_F))
